# baseline (speedup 1.0000x reference)
import functools
import math

import jax
import jax.numpy as jnp
from jax import lax
from jax.experimental import pallas as pl
from jax.experimental.pallas import tpu as pltpu
from jax.experimental.pallas import tpu_sc as plsc

_B = 16384
_D = 64
_EPS = 1e-5

_NC = 2
_NS = 16
_NW = _NC * _NS
_PS = _B // _NS
_CH = 128
_NR = _PS // _CH


def _gather_body(utab, itab, pids, uout, iout, pid_v, pair_v, sem, osem):
    wid = lax.axis_index("s") * _NC + lax.axis_index("c")
    pltpu.sync_copy(pids.at[wid], pid_v)

    def do_table(tab, out_hbm, obase):
        def round_body(r, carry):
            pltpu.async_copy(tab.at[pid_v.at[r]], pair_v, sem).wait()
            pltpu.async_copy(
                pair_v, out_hbm.at[pl.ds(obase + r * _CH, _CH)], osem).wait()
            return carry
        lax.fori_loop(0, _NR, round_body, 0)

    @pl.when(wid < _NS)
    def _():
        do_table(utab, uout, wid * _PS)

    @pl.when(wid >= _NS)
    def _():
        do_table(itab, iout, (wid - _NS) * _PS)


@functools.cache
def _gather2():
    return pl.kernel(
        _gather_body,
        mesh=plsc.VectorSubcoreMesh(core_axis_name="c", subcore_axis_name="s"),
        out_type=(
            jax.ShapeDtypeStruct((_B, 2 * _D), jnp.float32),
            jax.ShapeDtypeStruct((_B, 2 * _D), jnp.float32),
        ),
        scratch_types=[
            pltpu.VMEM((_NR, _CH), jnp.int32),
            pltpu.VMEM((_CH, 2 * _D), jnp.float32),
            pltpu.SemaphoreType.DMA,
            pltpu.SemaphoreType.DMA,
        ],
        compiler_params=pltpu.CompilerParams(use_tc_tiling_on_sc=False),
    )


_BM = 1024


def _dense_body(t_ref, d_ref, up_ref, ip_ref, mu_ref, mi_ref,
                ct_ref, cd_ref, au_ref, ai_ref, c1_ref,
                a2_ref, c2_ref, a3_ref, c3_ref, o_ref):
    up, ip, mu, mi = up_ref[...], ip_ref[...], mu_ref[...], mi_ref[...]
    ue = up[:, :_D] + mu * (up[:, _D:] - up[:, :_D])
    ie = ip[:, :_D] + mi * (ip[:, _D:] - ip[:, :_D])
    h1 = jnp.dot(t_ref[...], ct_ref[...], preferred_element_type=jnp.float32)
    h1 += jnp.dot(d_ref[...], cd_ref[...], preferred_element_type=jnp.float32)
    h1 += jnp.dot(ue, au_ref[...], preferred_element_type=jnp.float32)
    h1 += jnp.dot(ie, ai_ref[...], preferred_element_type=jnp.float32)
    h1 = jnp.maximum(h1 + c1_ref[...], 0.0)
    h2 = jnp.maximum(
        jnp.dot(h1, a2_ref[...], preferred_element_type=jnp.float32)
        + c2_ref[...], 0.0)
    o_ref[...] = (jnp.dot(h2, a3_ref[...], preferred_element_type=jnp.float32)
                  + c3_ref[...])


def kernel(user_ids, item_ids, title_embeddings, description_embeddings,
           title_embeddings_user_avg, description_embeddings_user_avg,
           user_table, item_table, Wt, bt, Wd, bd,
           W1, b1, W2, b2, W3, b3, g1, be1, g2, be2, g3, be3):
    uid = user_ids.astype(jnp.int32)
    iid = item_ids.astype(jnp.int32)
    pids = jnp.concatenate([uid >> 1, iid >> 1]).reshape(_NW, _NR, _CH)
    mu = (uid & 1).astype(jnp.float32)[:, None]
    mi = (iid & 1).astype(jnp.float32)[:, None]
    utab2 = user_table.reshape(-1, 2 * _D)
    itab2 = item_table.reshape(-1, 2 * _D)
    upair, ipair = _gather2()(utab2, itab2, pids)

    s = 1.0 / math.sqrt(1.0 + _EPS)
    w1e = W1 * (s * g1)[None, :]
    b1e = b1 + be1 @ W1.T
    w1u, w1i = w1e[:, :_D], w1e[:, _D:2 * _D]
    w1t, w1d = w1e[:, 2 * _D:3 * _D], w1e[:, 3 * _D:]
    ct = (w1t @ Wt).T
    cd = (w1d @ Wd).T
    c1 = (b1e + bt @ w1t.T + bd @ w1d.T)[None, :]
    au, ai = w1u.T, w1i.T
    a2 = (W2 * (s * g2)[None, :]).T
    c2 = (b2 + be2 @ W2.T)[None, :]
    a3 = (W3 * (s * g3)[None, :]).T
    c3 = (b3 + be3 @ W3.T)[None, :]

    grid = (_B // _BM,)
    full = lambda shape: pl.BlockSpec(shape, lambda i: (0, 0))
    out2d = pl.pallas_call(
        _dense_body,
        grid=grid,
        in_specs=[
            pl.BlockSpec((_BM, 768), lambda i: (i, 0)),
            pl.BlockSpec((_BM, 768), lambda i: (i, 0)),
            pl.BlockSpec((_BM, 2 * _D), lambda i: (i, 0)),
            pl.BlockSpec((_BM, 2 * _D), lambda i: (i, 0)),
            pl.BlockSpec((_BM, 1), lambda i: (i, 0)),
            pl.BlockSpec((_BM, 1), lambda i: (i, 0)),
            full((768, 128)), full((768, 128)),
            full((_D, 128)), full((_D, 128)), full((1, 128)),
            full((128, _D)), full((1, _D)),
            full((_D, 1)), full((1, 1)),
        ],
        out_specs=pl.BlockSpec((_BM, 1), lambda i: (i, 0)),
        out_shape=jax.ShapeDtypeStruct((_B, 1), jnp.float32),
    )(title_embeddings_user_avg, description_embeddings_user_avg,
      upair, ipair, mu, mi, ct, cd, au, ai, c1, a2, c2, a3, c3)
    return out2d[:, 0]
